# Initial kernel scaffold; baseline (speedup 1.0000x reference)
import jax
import jax.numpy as jnp
from jax.experimental import pallas as pl


def kernel(rois, pts, pts_feature):
    raise NotImplementedError("write your pallas kernel here")


# speedup vs baseline: 15.0207x; 15.0207x over previous
import jax
import jax.numpy as jnp
from jax import lax
from jax.experimental import pallas as pl
from jax.experimental.pallas import tpu as pltpu
from jax.experimental.pallas import tpu_sc as plsc

NROI = 64
NPTS = 20000
NFEAT = 16
OX = OY = OZ = 12
NVOX = OX * OY * OZ
MV = 128
NWORK = 32
RPW = NROI // NWORK
PVEC = NPTS // 16


def _body(xs, ys, zs, roip, feat, outf, outsel,
          xs_v, ys_v, zs_v, roip_v, pool_v, occ_v, sel_v,
          row_v, outf_v, outsel_v):
  cid = lax.axis_index("c")
  sid = lax.axis_index("s")
  wid = sid * 2 + cid

  pltpu.sync_copy(xs, xs_v)
  pltpu.sync_copy(ys, ys_v)
  pltpu.sync_copy(zs, zs_v)
  pltpu.sync_copy(roip, roip_v)

  for r in range(RPW):
    n = wid * RPW + r
    prm = roip_v[pl.ds(n * 16, 16)]
    cx = prm[0]
    cy = prm[1]
    cz = prm[2]
    cc = prm[3]
    ss = prm[4]
    dx = prm[5]
    dy = prm[6]
    dz = prm[7]
    gx = prm[8]
    gy = prm[9]
    gz = prm[10]
    hx = prm[11]
    hy = prm[12]

    def zocc(i, _):
      occ_v[pl.ds(i * 16, 16)] = jnp.zeros((16,), jnp.int32)
      return 0
    lax.fori_loop(0, NVOX // 16, zocc, 0)

    neg_inf = jnp.full((16,), -jnp.inf, jnp.float32)

    def zpool(i, _):
      pool_v[pl.ds(i * 16, 16)] = neg_inf
      return 0
    lax.fori_loop(0, NVOX * NFEAT // 16, zpool, 0)

    ones16 = jnp.ones((16,), jnp.int32)

    def sweep(i, _):
      x = xs_v[pl.ds(i * 16, 16)]
      y = ys_v[pl.ds(i * 16, 16)]
      z = zs_v[pl.ds(i * 16, 16)]
      sx = x - cx
      sy = y - cy
      xl = sx * cc - sy * ss + hx
      yl = sx * ss + sy * cc + hy
      zl = z - cz
      inb = ((xl >= 0.0) & (xl < dx) & (yl >= 0.0) & (yl < dy)
             & (zl >= 0.0) & (zl < dz))
      mi = inb.astype(jnp.int32)
      cnt = jnp.sum(mi)

      @pl.when(cnt > 0)
      def _():
        vx = jnp.clip((xl / gx).astype(jnp.int32), 0, OX - 1)
        vy = jnp.clip((yl / gy).astype(jnp.int32), 0, OY - 1)
        vz = jnp.clip((zl / gz).astype(jnp.int32), 0, OZ - 1)
        vox = (vx * OY + vy) * OZ + vz
        plsc.store_scatter(occ_v, [vox], ones16, mask=inb)
        for l in range(16):
          @pl.when(mi[l] != 0)
          def _():
            base = vox[l] * NFEAT
            pltpu.sync_copy(feat.at[i * 16 + l], row_v)
            fr = row_v[...]
            pool_v[pl.ds(base, 16)] = jnp.maximum(pool_v[pl.ds(base, 16)], fr)
      return 0
    lax.fori_loop(0, PVEC, sweep, 0)

    def compact(i, pos):
      ov = occ_v[pl.ds(i * 16, 16)]
      m = ov != 0
      mi2 = m.astype(jnp.int32)
      ids = lax.iota(jnp.int32, 16) + i * 16
      tgt = pos + (plsc.cumsum(mi2) - mi2)
      plsc.store_scatter(sel_v, [tgt], ids, mask=m)
      return pos + jnp.sum(mi2)
    n_ne = lax.fori_loop(0, NVOX // 16, compact, 0)

    jv16 = lax.iota(jnp.int32, 16)

    def emit(jv, _):
      selvec = sel_v[pl.ds(jv * 16, 16)]
      validv = (jv16 + jv * 16) < n_ne
      validi = validv.astype(jnp.int32)
      safe = jnp.where(validv, selvec, 0)
      outsel_v[pl.ds(jv * 16, 16)] = jnp.where(validv, selvec, -1)
      for l in range(16):
        rowd = pool_v[pl.ds(safe[l] * NFEAT, 16)]
        outf_v[pl.ds((jv * 16 + l) * 16, 16)] = jnp.where(
            validi[l] != 0, rowd, 0.0)
      return 0
    lax.fori_loop(0, MV // 16, emit, 0)

    pltpu.sync_copy(outf_v, outf.at[n])
    pltpu.sync_copy(outsel_v, outsel.at[n])


@jax.jit
def _run(xs, ys, zs, roip, feat):
  f = pl.kernel(
      _body,
      out_type=(jax.ShapeDtypeStruct((NROI, MV * NFEAT), jnp.float32),
                jax.ShapeDtypeStruct((NROI, MV), jnp.int32)),
      mesh=plsc.VectorSubcoreMesh(core_axis_name="c", subcore_axis_name="s"),
      compiler_params=pltpu.CompilerParams(needs_layout_passes=False),
      scratch_types=[
          pltpu.VMEM((NPTS,), jnp.float32),
          pltpu.VMEM((NPTS,), jnp.float32),
          pltpu.VMEM((NPTS,), jnp.float32),
          pltpu.VMEM((NROI * 16,), jnp.float32),
          pltpu.VMEM((NVOX * NFEAT,), jnp.float32),
          pltpu.VMEM((NVOX,), jnp.int32),
          pltpu.VMEM((NVOX + 16,), jnp.int32),
          pltpu.VMEM((16,), jnp.float32),
          pltpu.VMEM((MV * NFEAT,), jnp.float32),
          pltpu.VMEM((MV,), jnp.int32),
      ],
  )
  return f(xs, ys, zs, roip, feat)


def kernel(rois, pts, pts_feature):
  centers = rois[:, 0:3]
  dims = rois[:, 3:6]
  rz = rois[:, 6]
  cc = jnp.cos(-rz)
  ss = jnp.sin(-rz)
  pad = jnp.zeros((NROI,), jnp.float32)
  roip = jnp.stack([
      centers[:, 0], centers[:, 1], centers[:, 2],
      cc, ss,
      dims[:, 0], dims[:, 1], dims[:, 2],
      dims[:, 0] / OX, dims[:, 1] / OY, dims[:, 2] / OZ,
      dims[:, 0] * 0.5, dims[:, 1] * 0.5,
      pad, pad, pad,
  ], axis=1).astype(jnp.float32).reshape(NROI * 16)
  xs = pts[:, 0].astype(jnp.float32)
  ys = pts[:, 1].astype(jnp.float32)
  zs = pts[:, 2].astype(jnp.float32)
  featout, selout = _run(xs, ys, zs, roip, pts_feature.astype(jnp.float32))
  pooled_features = featout.reshape(NROI, MV, NFEAT)
  valid = selout >= 0
  svx = selout // (OY * OZ)
  rem = selout % (OY * OZ)
  svy = rem // OZ
  svz = rem % OZ
  coors = jnp.stack([svx, svy, svz], axis=-1).astype(jnp.int32)
  pooled_coors = jnp.where(valid[..., None], coors, -1)
  return pooled_features, pooled_coors


# speedup vs baseline: 15.9449x; 1.0615x over previous
import jax
import jax.numpy as jnp
from jax import lax
from jax.experimental import pallas as pl
from jax.experimental.pallas import tpu as pltpu
from jax.experimental.pallas import tpu_sc as plsc

NROI = 64
NPTS = 20000
NFEAT = 16
OX = OY = OZ = 12
NVOX = OX * OY * OZ
MV = 128
NWORK = 32
RPW = NROI // NWORK
PVEC = NPTS // 16


def _body(xs, ys, zs, roip, feat, outf, outsel,
          xs_v, ys_v, zs_v, roip_v, pool_v, occ_v, sel_v,
          row_v, outf_v, outsel_v):
  cid = lax.axis_index("c")
  sid = lax.axis_index("s")
  wid = sid * 2 + cid

  pltpu.sync_copy(xs, xs_v)
  pltpu.sync_copy(ys, ys_v)
  pltpu.sync_copy(zs, zs_v)
  pltpu.sync_copy(roip, roip_v)

  for r in range(RPW):
    n = wid * RPW + r
    prm = roip_v[pl.ds(n * 16, 16)]
    cx = prm[0]
    cy = prm[1]
    cz = prm[2]
    cc = prm[3]
    ss = prm[4]
    dx = prm[5]
    dy = prm[6]
    dz = prm[7]
    gx = prm[8]
    gy = prm[9]
    gz = prm[10]
    hx = prm[11]
    hy = prm[12]

    zero16 = jnp.zeros((16,), jnp.int32)

    def zocc(i, _):
      for u in range(4):
        occ_v[pl.ds((i * 4 + u) * 16, 16)] = zero16
      return 0
    lax.fori_loop(0, NVOX // 64, zocc, 0)

    neg_inf = jnp.full((16,), -jnp.inf, jnp.float32)

    def zpool(i, _):
      for u in range(4):
        pool_v[pl.ds((i * 4 + u) * 16, 16)] = neg_inf
      return 0
    lax.fori_loop(0, NVOX * NFEAT // 64, zpool, 0)

    ones16 = jnp.ones((16,), jnp.int32)
    cx_v = jnp.broadcast_to(cx, (16,))
    cy_v = jnp.broadcast_to(cy, (16,))
    cz_v = jnp.broadcast_to(cz, (16,))
    cc_v = jnp.broadcast_to(cc, (16,))
    ss_v = jnp.broadcast_to(ss, (16,))
    dx_v = jnp.broadcast_to(dx, (16,))
    dy_v = jnp.broadcast_to(dy, (16,))
    dz_v = jnp.broadcast_to(dz, (16,))
    gx_v = jnp.broadcast_to(gx, (16,))
    gy_v = jnp.broadcast_to(gy, (16,))
    gz_v = jnp.broadcast_to(gz, (16,))
    hx_v = jnp.broadcast_to(hx, (16,))
    hy_v = jnp.broadcast_to(hy, (16,))
    zero16f = jnp.zeros((16,), jnp.float32)

    def sweep(i, _):
      x = xs_v[pl.ds(i * 16, 16)]
      y = ys_v[pl.ds(i * 16, 16)]
      z = zs_v[pl.ds(i * 16, 16)]
      sx = x - cx_v
      sy = y - cy_v
      xl = sx * cc_v - sy * ss_v + hx_v
      yl = sx * ss_v + sy * cc_v + hy_v
      zl = z - cz_v
      inb = ((xl >= zero16f) & (xl < dx_v) & (yl >= zero16f) & (yl < dy_v)
             & (zl >= zero16f) & (zl < dz_v))
      cnt = plsc.all_reduce_population_count(inb)[0]

      @pl.when(cnt > 0)
      def _():
        mi = inb.astype(jnp.int32)
        vx = jnp.clip((xl / gx_v).astype(jnp.int32), 0, OX - 1)
        vy = jnp.clip((yl / gy_v).astype(jnp.int32), 0, OY - 1)
        vz = jnp.clip((zl / gz_v).astype(jnp.int32), 0, OZ - 1)
        vox = (vx * OY + vy) * OZ + vz
        plsc.store_scatter(occ_v, [vox], ones16, mask=inb)
        for l in range(16):
          @pl.when(mi[l] != 0)
          def _():
            base = vox[l] * NFEAT
            pltpu.sync_copy(feat.at[i * 16 + l], row_v)
            fr = row_v[...]
            pool_v[pl.ds(base, 16)] = jnp.maximum(pool_v[pl.ds(base, 16)], fr)
      return 0
    lax.fori_loop(0, PVEC, sweep, 0)

    def compact(i, pos):
      ov = occ_v[pl.ds(i * 16, 16)]
      m = ov != 0
      mi2 = m.astype(jnp.int32)
      ids = lax.iota(jnp.int32, 16) + i * 16
      tgt = pos + (plsc.cumsum(mi2) - mi2)
      plsc.store_scatter(sel_v, [tgt], ids, mask=m)
      return pos + plsc.all_reduce_population_count(m)[0]
    n_ne = lax.fori_loop(0, NVOX // 16, compact, 0)

    jv16 = lax.iota(jnp.int32, 16)

    def emit(jv, _):
      selvec = sel_v[pl.ds(jv * 16, 16)]
      validv = (jv16 + jv * 16) < n_ne
      validi = validv.astype(jnp.int32)
      safe = jnp.where(validv, selvec, 0)
      outsel_v[pl.ds(jv * 16, 16)] = jnp.where(validv, selvec, -1)
      for l in range(16):
        rowd = pool_v[pl.ds(safe[l] * NFEAT, 16)]
        outf_v[pl.ds((jv * 16 + l) * 16, 16)] = jnp.where(
            validi[l] != 0, rowd, 0.0)
      return 0
    lax.fori_loop(0, MV // 16, emit, 0)

    pltpu.sync_copy(outf_v, outf.at[n])
    pltpu.sync_copy(outsel_v, outsel.at[n])


@jax.jit
def _run(xs, ys, zs, roip, feat):
  f = pl.kernel(
      _body,
      out_type=(jax.ShapeDtypeStruct((NROI, MV * NFEAT), jnp.float32),
                jax.ShapeDtypeStruct((NROI, MV), jnp.int32)),
      mesh=plsc.VectorSubcoreMesh(core_axis_name="c", subcore_axis_name="s"),
      compiler_params=pltpu.CompilerParams(needs_layout_passes=False),
      scratch_types=[
          pltpu.VMEM((NPTS,), jnp.float32),
          pltpu.VMEM((NPTS,), jnp.float32),
          pltpu.VMEM((NPTS,), jnp.float32),
          pltpu.VMEM((NROI * 16,), jnp.float32),
          pltpu.VMEM((NVOX * NFEAT,), jnp.float32),
          pltpu.VMEM((NVOX,), jnp.int32),
          pltpu.VMEM((NVOX + 16,), jnp.int32),
          pltpu.VMEM((16,), jnp.float32),
          pltpu.VMEM((MV * NFEAT,), jnp.float32),
          pltpu.VMEM((MV,), jnp.int32),
      ],
  )
  return f(xs, ys, zs, roip, feat)


def kernel(rois, pts, pts_feature):
  centers = rois[:, 0:3]
  dims = rois[:, 3:6]
  rz = rois[:, 6]
  cc = jnp.cos(-rz)
  ss = jnp.sin(-rz)
  pad = jnp.zeros((NROI,), jnp.float32)
  roip = jnp.stack([
      centers[:, 0], centers[:, 1], centers[:, 2],
      cc, ss,
      dims[:, 0], dims[:, 1], dims[:, 2],
      dims[:, 0] / OX, dims[:, 1] / OY, dims[:, 2] / OZ,
      dims[:, 0] * 0.5, dims[:, 1] * 0.5,
      pad, pad, pad,
  ], axis=1).astype(jnp.float32).reshape(NROI * 16)
  xs = pts[:, 0].astype(jnp.float32)
  ys = pts[:, 1].astype(jnp.float32)
  zs = pts[:, 2].astype(jnp.float32)
  featout, selout = _run(xs, ys, zs, roip, pts_feature.astype(jnp.float32))
  pooled_features = featout.reshape(NROI, MV, NFEAT)
  valid = selout >= 0
  svx = selout // (OY * OZ)
  rem = selout % (OY * OZ)
  svy = rem // OZ
  svz = rem % OZ
  coors = jnp.stack([svx, svy, svz], axis=-1).astype(jnp.int32)
  pooled_coors = jnp.where(valid[..., None], coors, -1)
  return pooled_features, pooled_coors


# speedup vs baseline: 17.4375x; 1.0936x over previous
import jax
import jax.numpy as jnp
from jax import lax
from jax.experimental import pallas as pl
from jax.experimental.pallas import tpu as pltpu
from jax.experimental.pallas import tpu_sc as plsc

NROI = 64
NPTS = 20000
NFEAT = 16
OX = OY = OZ = 12
NVOX = OX * OY * OZ
MV = 128
NWORK = 32
RPW = NROI // NWORK
PVEC = NPTS // 16


def _broadcast_params(prm):
  return [jnp.broadcast_to(prm[k], (16,)) for k in range(13)]


def _body(xs, ys, zs, roip, feat, outf, outsel,
          xs_v, ys_v, zs_v, roip_v, pool0_v, pool1_v, occ0_v, occ1_v,
          sel_v, row_v, outf_v, outsel_v):
  cid = lax.axis_index("c")
  sid = lax.axis_index("s")
  wid = sid * 2 + cid

  pltpu.sync_copy(xs, xs_v)
  pltpu.sync_copy(ys, ys_v)
  pltpu.sync_copy(zs, zs_v)
  pltpu.sync_copy(roip, roip_v)

  n0 = wid * RPW
  n1 = n0 + 1
  (cx0, cy0, cz0, cc0, ss0, dx0, dy0, dz0, gx0, gy0, gz0, hx0, hy0
   ) = _broadcast_params(roip_v[pl.ds(n0 * 16, 16)])
  (cx1, cy1, cz1, cc1, ss1, dx1, dy1, dz1, gx1, gy1, gz1, hx1, hy1
   ) = _broadcast_params(roip_v[pl.ds(n1 * 16, 16)])

  zero16 = jnp.zeros((16,), jnp.int32)
  neg_inf = jnp.full((16,), -jnp.inf, jnp.float32)
  ones16 = jnp.ones((16,), jnp.int32)
  zero16f = jnp.zeros((16,), jnp.float32)

  def zocc(i, _):
    for u in range(2):
      occ0_v[pl.ds((i * 2 + u) * 16, 16)] = zero16
      occ1_v[pl.ds((i * 2 + u) * 16, 16)] = zero16
    return 0
  lax.fori_loop(0, NVOX // 32, zocc, 0)

  def zpool(i, _):
    for u in range(2):
      pool0_v[pl.ds((i * 2 + u) * 16, 16)] = neg_inf
      pool1_v[pl.ds((i * 2 + u) * 16, 16)] = neg_inf
    return 0
  lax.fori_loop(0, NVOX * NFEAT // 32, zpool, 0)

  def sweep(i, _):
    x = xs_v[pl.ds(i * 16, 16)]
    y = ys_v[pl.ds(i * 16, 16)]
    z = zs_v[pl.ds(i * 16, 16)]
    sx0 = x - cx0
    sy0 = y - cy0
    xl0 = sx0 * cc0 - sy0 * ss0 + hx0
    yl0 = sx0 * ss0 + sy0 * cc0 + hy0
    zl0 = z - cz0
    inb0 = (((xl0 >= zero16f) & (xl0 < dx0))
            & ((yl0 >= zero16f) & (yl0 < dy0))
            & ((zl0 >= zero16f) & (zl0 < dz0)))
    sx1 = x - cx1
    sy1 = y - cy1
    xl1 = sx1 * cc1 - sy1 * ss1 + hx1
    yl1 = sx1 * ss1 + sy1 * cc1 + hy1
    zl1 = z - cz1
    inb1 = (((xl1 >= zero16f) & (xl1 < dx1))
            & ((yl1 >= zero16f) & (yl1 < dy1))
            & ((zl1 >= zero16f) & (zl1 < dz1)))
    hit = plsc.all_reduce_population_count(inb0 | inb1)[0]

    @pl.when(hit > 0)
    def _():
      cnt0 = plsc.all_reduce_population_count(inb0)[0]
      cnt1 = plsc.all_reduce_population_count(inb1)[0]

      @pl.when(cnt0 > 0)
      def _():
        mi = inb0.astype(jnp.int32)
        vx = jnp.clip((xl0 / gx0).astype(jnp.int32), 0, OX - 1)
        vy = jnp.clip((yl0 / gy0).astype(jnp.int32), 0, OY - 1)
        vz = jnp.clip((zl0 / gz0).astype(jnp.int32), 0, OZ - 1)
        vox = (vx * OY + vy) * OZ + vz
        plsc.store_scatter(occ0_v, [vox], ones16, mask=inb0)
        for l in range(16):
          @pl.when(mi[l] != 0)
          def _():
            base = vox[l] * NFEAT
            pltpu.sync_copy(feat.at[i * 16 + l], row_v)
            fr = row_v[...]
            pool0_v[pl.ds(base, 16)] = jnp.maximum(
                pool0_v[pl.ds(base, 16)], fr)

      @pl.when(cnt1 > 0)
      def _():
        mi = inb1.astype(jnp.int32)
        vx = jnp.clip((xl1 / gx1).astype(jnp.int32), 0, OX - 1)
        vy = jnp.clip((yl1 / gy1).astype(jnp.int32), 0, OY - 1)
        vz = jnp.clip((zl1 / gz1).astype(jnp.int32), 0, OZ - 1)
        vox = (vx * OY + vy) * OZ + vz
        plsc.store_scatter(occ1_v, [vox], ones16, mask=inb1)
        for l in range(16):
          @pl.when(mi[l] != 0)
          def _():
            base = vox[l] * NFEAT
            pltpu.sync_copy(feat.at[i * 16 + l], row_v)
            fr = row_v[...]
            pool1_v[pl.ds(base, 16)] = jnp.maximum(
                pool1_v[pl.ds(base, 16)], fr)
    return 0
  lax.fori_loop(0, PVEC, sweep, 0)

  jv16 = lax.iota(jnp.int32, 16)

  for n, pool_v, occ_v in ((n0, pool0_v, occ0_v), (n1, pool1_v, occ1_v)):
    def compact(i, pos):
      ov = occ_v[pl.ds(i * 16, 16)]
      m = ov != 0
      mi2 = m.astype(jnp.int32)
      ids = lax.iota(jnp.int32, 16) + i * 16
      tgt = pos + (plsc.cumsum(mi2) - mi2)
      plsc.store_scatter(sel_v, [tgt], ids, mask=m)
      return pos + plsc.all_reduce_population_count(m)[0]
    n_ne = lax.fori_loop(0, NVOX // 16, compact, 0)

    def emit(jv, _):
      selvec = sel_v[pl.ds(jv * 16, 16)]
      validv = (jv16 + jv * 16) < n_ne
      validi = validv.astype(jnp.int32)
      safe = jnp.where(validv, selvec, 0)
      outsel_v[pl.ds(jv * 16, 16)] = jnp.where(validv, selvec, -1)
      for l in range(16):
        rowd = pool_v[pl.ds(safe[l] * NFEAT, 16)]
        outf_v[pl.ds((jv * 16 + l) * 16, 16)] = jnp.where(
            validi[l] != 0, rowd, 0.0)
      return 0
    lax.fori_loop(0, MV // 16, emit, 0)

    pltpu.sync_copy(outf_v, outf.at[n])
    pltpu.sync_copy(outsel_v, outsel.at[n])


@jax.jit
def _run(xs, ys, zs, roip, feat):
  f = pl.kernel(
      _body,
      out_type=(jax.ShapeDtypeStruct((NROI, MV * NFEAT), jnp.float32),
                jax.ShapeDtypeStruct((NROI, MV), jnp.int32)),
      mesh=plsc.VectorSubcoreMesh(core_axis_name="c", subcore_axis_name="s"),
      compiler_params=pltpu.CompilerParams(needs_layout_passes=False),
      scratch_types=[
          pltpu.VMEM((NPTS,), jnp.float32),
          pltpu.VMEM((NPTS,), jnp.float32),
          pltpu.VMEM((NPTS,), jnp.float32),
          pltpu.VMEM((NROI * 16,), jnp.float32),
          pltpu.VMEM((NVOX * NFEAT,), jnp.float32),
          pltpu.VMEM((NVOX * NFEAT,), jnp.float32),
          pltpu.VMEM((NVOX,), jnp.int32),
          pltpu.VMEM((NVOX,), jnp.int32),
          pltpu.VMEM((NVOX + 16,), jnp.int32),
          pltpu.VMEM((16,), jnp.float32),
          pltpu.VMEM((MV * NFEAT,), jnp.float32),
          pltpu.VMEM((MV,), jnp.int32),
      ],
  )
  return f(xs, ys, zs, roip, feat)


def kernel(rois, pts, pts_feature):
  centers = rois[:, 0:3]
  dims = rois[:, 3:6]
  rz = rois[:, 6]
  cc = jnp.cos(-rz)
  ss = jnp.sin(-rz)
  pad = jnp.zeros((NROI,), jnp.float32)
  roip = jnp.stack([
      centers[:, 0], centers[:, 1], centers[:, 2],
      cc, ss,
      dims[:, 0], dims[:, 1], dims[:, 2],
      dims[:, 0] / OX, dims[:, 1] / OY, dims[:, 2] / OZ,
      dims[:, 0] * 0.5, dims[:, 1] * 0.5,
      pad, pad, pad,
  ], axis=1).astype(jnp.float32).reshape(NROI * 16)
  xs = pts[:, 0].astype(jnp.float32)
  ys = pts[:, 1].astype(jnp.float32)
  zs = pts[:, 2].astype(jnp.float32)
  featout, selout = _run(xs, ys, zs, roip, pts_feature.astype(jnp.float32))
  pooled_features = featout.reshape(NROI, MV, NFEAT)
  valid = selout >= 0
  svx = selout // (OY * OZ)
  rem = selout % (OY * OZ)
  svy = rem // OZ
  svz = rem % OZ
  coors = jnp.stack([svx, svy, svz], axis=-1).astype(jnp.int32)
  pooled_coors = jnp.where(valid[..., None], coors, -1)
  return pooled_features, pooled_coors
